# baseline (speedup 1.0000x reference)
import functools

import jax
import jax.numpy as jnp
from jax import lax
from jax.experimental import pallas as pl
from jax.experimental.pallas import tpu as pltpu
from jax.experimental.pallas import tpu_sc as plsc

_LW = 128


def _argmin_body(a_ref, z2_ref, cb_ref, cn_ref, ind_ref, loss_ref, mm_scr):
    i = pl.program_id(0)
    br = z2_ref.shape[0]
    e = cb_ref.shape[1]
    g_cnt = e // _LW
    rs = min(64, br)
    nw_ = min(1024, e)
    z2 = z2_ref[...] + z2_ref[...]
    for q in range(e // nw_):
        mm_scr[:, q * nw_:(q + 1) * nw_] = jnp.dot(
            z2, cb_ref[:, q * nw_:(q + 1) * nw_],
            preferred_element_type=jnp.float32)
    cn = cn_ref[...]
    lane = lax.broadcasted_iota(jnp.int32, (rs, _LW), 1).astype(jnp.float32)
    tot = jnp.float32(0.0)
    jmins = []
    for r in range(br // rs):
        ab = jnp.broadcast_to(a_ref[r * rs:(r + 1) * rs, :], (rs, _LW))
        rv = (ab - mm_scr[r * rs:(r + 1) * rs, 0:_LW]) + jnp.broadcast_to(
            cn[:, 0:_LW], (rs, _LW))
        rg = jnp.zeros((rs, _LW), jnp.float32)
        for g in range(1, g_cnt):
            sl = slice(g * _LW, (g + 1) * _LW)
            cur = (ab - mm_scr[r * rs:(r + 1) * rs, sl]) + jnp.broadcast_to(
                cn[:, sl], (rs, _LW))
            upd = cur < rv
            rv = jnp.where(upd, cur, rv)
            rg = jnp.where(upd, jnp.float32(g), rg)
        rowmin = jnp.min(rv, axis=1, keepdims=True)
        lanej = rg * jnp.float32(_LW) + lane
        cand = jnp.where(rv == rowmin, lanej, jnp.float32(3e9))
        jmin = jnp.min(cand, axis=1, keepdims=True)
        jmins.append(jmin.astype(jnp.int32))
        tot = tot + jnp.sum(rowmin)
    ind_ref[...] = jnp.reshape(jnp.concatenate(jmins, axis=0),
                               (br // _LW, _LW))

    @pl.when(i == 0)
    def _():
        loss_ref[0, 0] = tot

    @pl.when(i > 0)
    def _():
        loss_ref[0, 0] = loss_ref[0, 0] + tot


def _argmin_call(a, z2, cb, cn, br, rows, blk0):
    n, d = z2.shape
    e = cb.shape[1]
    grid = (rows // br,)
    return pl.pallas_call(
        _argmin_body,
        grid=grid,
        in_specs=[
            pl.BlockSpec((br, 1), lambda i: (i + blk0, 0)),
            pl.BlockSpec((br, d), lambda i: (i + blk0, 0)),
            pl.BlockSpec((d, e), lambda i: (0, 0)),
            pl.BlockSpec((1, e), lambda i: (0, 0)),
        ],
        out_specs=[
            pl.BlockSpec((br // _LW, _LW), lambda i: (i, 0)),
            pl.BlockSpec((1, 1), lambda i: (0, 0),
                         memory_space=pltpu.SMEM),
        ],
        out_shape=[
            jax.ShapeDtypeStruct((rows // _LW, _LW), jnp.int32),
            jax.ShapeDtypeStruct((1, 1), jnp.float32),
        ],
        scratch_shapes=[
            pltpu.VMEM((br, e), jnp.float32),
        ],
    )(a, z2, cb, cn)


def _gather_body(nc, ch, nchunk, table_hbm, idx_hbm, out_hbm,
                 idx0, idx1, rows0, rows1, g0, g1, w0, w1):
    wid = lax.axis_index("s") * nc + lax.axis_index("c")
    idxb = (idx0, idx1)
    rowsb = (rows0, rows1)
    gsem = (g0, g1)
    wsem = (w0, w1)
    base = wid * nchunk
    pltpu.sync_copy(idx_hbm.at[base], idx0)
    gcop = [pltpu.async_copy(table_hbm.at[idx0], rows0, g0), None]
    wcop = [None, None]
    for t in range(nchunk):
        b = t % 2
        nb = (t + 1) % 2
        if t + 1 < nchunk:
            if wcop[nb] is not None:
                wcop[nb].wait()
            pltpu.sync_copy(idx_hbm.at[base + t + 1], idxb[nb])
            gcop[nb] = pltpu.async_copy(table_hbm.at[idxb[nb]], rowsb[nb],
                                        gsem[nb])
        gcop[b].wait()
        wcop[b] = pltpu.async_copy(rowsb[b],
                                   out_hbm.at[pl.ds((base + t) * ch, ch)],
                                   wsem[b])
    for b in (0, 1):
        if wcop[b] is not None:
            wcop[b].wait()


def _gather_call(table, idx2d):
    v, d = table.shape
    nrow, ch = idx2d.shape
    b = nrow * ch
    info = plsc.get_sparse_core_info()
    nw = info.num_cores * info.num_subcores
    nchunk = nrow // nw
    mesh = plsc.VectorSubcoreMesh(core_axis_name="c", subcore_axis_name="s")
    k = functools.partial(
        pl.kernel,
        mesh=mesh,
        out_type=jax.ShapeDtypeStruct((b, d), jnp.float32),
        scratch_types=[
            pltpu.VMEM((ch,), jnp.int32),
            pltpu.VMEM((ch,), jnp.int32),
            pltpu.VMEM((ch, d), jnp.float32),
            pltpu.VMEM((ch, d), jnp.float32),
            pltpu.SemaphoreType.DMA,
            pltpu.SemaphoreType.DMA,
            pltpu.SemaphoreType.DMA,
            pltpu.SemaphoreType.DMA,
        ],
    )(functools.partial(_gather_body, info.num_cores, ch, nchunk))
    return k(table, idx2d)


def kernel(latents, codebook):
    n, d = latents.shape
    a = jnp.sum(latents ** 2, axis=1, keepdims=True)
    cn = jnp.sum(codebook ** 2, axis=0, keepdims=True)
    table = codebook.T
    h = n // 2
    br = min(1024, h)
    parts = []
    for s_ in range(2):
        ind2d_h, loss_h = _argmin_call(a, latents, codebook, cn,
                                       br=br, rows=h, blk0=s_ * (h // br))
        q_h = _gather_call(table, ind2d_h)
        parts.append((ind2d_h, loss_h, q_h))
    ind = jnp.concatenate([p[0] for p in parts], axis=0).reshape(n)
    quantize = jnp.concatenate([p[2] for p in parts], axis=0)
    vq_loss = (parts[0][1][0, 0] + parts[1][1][0, 0]) / jnp.float32(n * d)
    return quantize, vq_loss, ind


# speedup vs baseline: 1.0852x; 1.0852x over previous
import functools

import jax
import jax.numpy as jnp
from jax import lax
from jax.experimental import pallas as pl
from jax.experimental.pallas import tpu as pltpu
from jax.experimental.pallas import tpu_sc as plsc

_LW = 128


def _argmin_body(a_ref, z2_ref, cb_ref, cn_ref, ind_ref, loss_ref, mm_scr):
    i = pl.program_id(0)
    br = z2_ref.shape[0]
    e = cb_ref.shape[1]
    g_cnt = e // _LW
    rs = min(64, br)
    nw_ = min(1024, e)
    z2 = z2_ref[...] + z2_ref[...]
    for q in range(e // nw_):
        mm_scr[:, q * nw_:(q + 1) * nw_] = jnp.dot(
            z2, cb_ref[:, q * nw_:(q + 1) * nw_],
            preferred_element_type=jnp.float32)
    cn = cn_ref[...]
    lane = lax.broadcasted_iota(jnp.int32, (rs, _LW), 1).astype(jnp.float32)
    tot = jnp.float32(0.0)
    for r in range(br // rs):
        ab = jnp.broadcast_to(a_ref[r * rs:(r + 1) * rs, :], (rs, _LW))
        rv = (ab - mm_scr[r * rs:(r + 1) * rs, 0:_LW]) + jnp.broadcast_to(
            cn[:, 0:_LW], (rs, _LW))
        rg = jnp.zeros((rs, _LW), jnp.float32)
        for g in range(1, g_cnt):
            sl = slice(g * _LW, (g + 1) * _LW)
            cur = (ab - mm_scr[r * rs:(r + 1) * rs, sl]) + jnp.broadcast_to(
                cn[:, sl], (rs, _LW))
            upd = cur < rv
            rv = jnp.where(upd, cur, rv)
            rg = jnp.where(upd, jnp.float32(g), rg)
        rowmin = jnp.min(rv, axis=1, keepdims=True)
        lanej = rg * jnp.float32(_LW) + lane
        cand = jnp.where(rv == rowmin, lanej, jnp.float32(3e9))
        jmin = jnp.min(cand, axis=1, keepdims=True)
        ind_ref[r * rs:(r + 1) * rs, :] = jmin.astype(jnp.int32)
        tot = tot + jnp.sum(rowmin)

    @pl.when(i == 0)
    def _():
        loss_ref[0, 0] = tot

    @pl.when(i > 0)
    def _():
        loss_ref[0, 0] = loss_ref[0, 0] + tot


def _argmin_call(a, z2, cb, cn, br):
    n, d = z2.shape
    e = cb.shape[1]
    grid = (n // br,)
    return pl.pallas_call(
        _argmin_body,
        grid=grid,
        in_specs=[
            pl.BlockSpec((br, 1), lambda i: (i, 0)),
            pl.BlockSpec((br, d), lambda i: (i, 0)),
            pl.BlockSpec((d, e), lambda i: (0, 0)),
            pl.BlockSpec((1, e), lambda i: (0, 0)),
        ],
        out_specs=[
            pl.BlockSpec((br, 1), lambda i: (i, 0)),
            pl.BlockSpec((1, 1), lambda i: (0, 0),
                         memory_space=pltpu.SMEM),
        ],
        out_shape=[
            jax.ShapeDtypeStruct((n, 1), jnp.int32),
            jax.ShapeDtypeStruct((1, 1), jnp.float32),
        ],
        scratch_shapes=[
            pltpu.VMEM((br, e), jnp.float32),
        ],
    )(a, z2, cb, cn)


def _gather_body(nc, ch, nchunk, table_hbm, idx_hbm, out_hbm,
                 idx0, idx1, rows0, rows1, g0, g1, w0, w1):
    wid = lax.axis_index("s") * nc + lax.axis_index("c")
    idxb = (idx0, idx1)
    rowsb = (rows0, rows1)
    gsem = (g0, g1)
    wsem = (w0, w1)
    base = wid * (ch * nchunk)
    pltpu.sync_copy(idx_hbm.at[pl.ds(base, ch)], idx0)
    gcop = [pltpu.async_copy(table_hbm.at[idx0], rows0, g0), None]
    wcop = [None, None]
    for t in range(nchunk):
        b = t % 2
        nb = (t + 1) % 2
        if t + 1 < nchunk:
            if wcop[nb] is not None:
                wcop[nb].wait()
            off = base + (t + 1) * ch
            pltpu.sync_copy(idx_hbm.at[pl.ds(off, ch)], idxb[nb])
            gcop[nb] = pltpu.async_copy(table_hbm.at[idxb[nb]], rowsb[nb],
                                        gsem[nb])
        gcop[b].wait()
        wcop[b] = pltpu.async_copy(rowsb[b],
                                   out_hbm.at[pl.ds(base + t * ch, ch)],
                                   wsem[b])
    for b in (0, 1):
        if wcop[b] is not None:
            wcop[b].wait()


def _gather_call(table, idx):
    v, d = table.shape
    b = idx.shape[0]
    info = plsc.get_sparse_core_info()
    nw = info.num_cores * info.num_subcores
    bpw = b // nw
    ch = min(bpw, 128)
    nchunk = bpw // ch
    mesh = plsc.VectorSubcoreMesh(core_axis_name="c", subcore_axis_name="s")
    k = functools.partial(
        pl.kernel,
        mesh=mesh,
        out_type=jax.ShapeDtypeStruct((b, d), jnp.float32),
        scratch_types=[
            pltpu.VMEM((ch,), jnp.int32),
            pltpu.VMEM((ch,), jnp.int32),
            pltpu.VMEM((ch, d), jnp.float32),
            pltpu.VMEM((ch, d), jnp.float32),
            pltpu.SemaphoreType.DMA,
            pltpu.SemaphoreType.DMA,
            pltpu.SemaphoreType.DMA,
            pltpu.SemaphoreType.DMA,
        ],
    )(functools.partial(_gather_body, info.num_cores, ch, nchunk))
    return k(table, idx)


def kernel(latents, codebook):
    n, d = latents.shape
    a = jnp.sum(latents ** 2, axis=1, keepdims=True)
    cn = jnp.sum(codebook ** 2, axis=0, keepdims=True)
    ind2d, loss_sum = _argmin_call(a, latents, codebook, cn,
                                   br=min(1024, n))
    ind = ind2d.reshape(n)
    quantize = _gather_call(codebook.T, ind)
    vq_loss = loss_sum[0, 0] / jnp.float32(n * d)
    return quantize, vq_loss, ind
